# baseline (speedup 1.0000x reference)
import functools

import jax
import jax.numpy as jnp
import numpy as np
from jax import lax
from jax.experimental import pallas as pl
from jax.experimental.pallas import tpu as pltpu
from jax.experimental.pallas import tpu_sc as plsc

B, S, D, V = 4096, 200, 64, 1000000
BS = B * S
OTHER = 64


_COLS_BLK = 32768
_N_BLKS = -(-V // _COLS_BLK)
_VPAD = _N_BLKS * _COLS_BLK


def _rowsum_body(emb_ref, out_ref):
    out_ref[...] = jnp.sum(emb_ref[...], axis=0)


_rowsum_call = pl.pallas_call(
    _rowsum_body,
    grid=(_N_BLKS,),
    in_specs=[pl.BlockSpec((D, _COLS_BLK), lambda i: (0, i))],
    out_specs=pl.BlockSpec((_COLS_BLK,), lambda i: (i,)),
    out_shape=jax.ShapeDtypeStruct((_VPAD,), jnp.float32),
)


_NC, _NS = 2, 16
_NW = _NC * _NS
_N_PER = BS // _NW


_N_CHUNKS = 4
_CHUNK = _N_PER // _N_CHUNKS


def _sc_gather_body(sums_hbm, idx_hbm, out_hbm, idx_v, vals_v, sems):
    wid = lax.axis_index("s") * _NC + lax.axis_index("c")
    base = wid * _N_PER
    copies = []
    for c in range(_N_CHUNKS):
        lo = c * _CHUNK
        pltpu.sync_copy(idx_hbm.at[pl.ds(base + lo, _CHUNK)],
                        idx_v.at[pl.ds(lo, _CHUNK)])
        copies.append(pltpu.async_copy(
            sums_hbm.at[idx_v.at[pl.ds(lo, _CHUNK)]],
            vals_v.at[pl.ds(lo, _CHUNK)], sems.at[c]))
    for c in range(_N_CHUNKS):
        lo = c * _CHUNK
        copies[c].wait()
        pltpu.sync_copy(vals_v.at[pl.ds(lo, _CHUNK)],
                        out_hbm.at[pl.ds(base + lo, _CHUNK)])


@functools.cache
def _sc_gather_call():
    return pl.kernel(
        _sc_gather_body,
        out_type=jax.ShapeDtypeStruct((BS,), jnp.float32),
        mesh=plsc.VectorSubcoreMesh(
            core_axis_name="c", subcore_axis_name="s", num_cores=_NC,
            num_subcores=_NS),
        scratch_types=[
            pltpu.VMEM((_N_PER,), jnp.int32),
            pltpu.VMEM((_N_PER,), jnp.float32),
            pltpu.SemaphoreType.DMA((_N_CHUNKS,)),
        ],
    )


_B_BLK = 512


def _tdot(a_t, w):
    return lax.dot_general(a_t, w, (((0,), (0,)), ((), ())),
                           preferred_element_type=jnp.float32)


def _mlp_body(g_ref, gwv_ref, gwov_ref, oth_ref, w1a_ref, w1b_ref, w1c_ref,
              w1d_ref, b1_ref, w2_ref, b2_ref, w3_ref, b3_ref, out_ref):
    gwv_t = gwv_ref[...]
    emb_t = g_ref[...] * gwv_t * (1.0 / D)
    h = (_tdot(emb_t, w1a_ref[...]) + _tdot(gwv_t, w1b_ref[...])
         + _tdot(gwov_ref[...], w1c_ref[...]) + _tdot(oth_ref[...], w1d_ref[...])
         + b1_ref[...])
    h = jnp.maximum(h, 0.0)
    h = jnp.maximum(
        jnp.dot(h, w2_ref[...], preferred_element_type=jnp.float32) + b2_ref[...],
        0.0)
    out_ref[...] = (
        jnp.dot(h, w3_ref[...], preferred_element_type=jnp.float32) + b3_ref[...])


def _full(shape):
    return pl.BlockSpec(shape, lambda i: tuple(0 for _ in shape))


_mlp_call = pl.pallas_call(
    _mlp_body,
    grid=(B // _B_BLK,),
    in_specs=[
        pl.BlockSpec((S, _B_BLK), lambda i: (0, i)),
        pl.BlockSpec((S, _B_BLK), lambda i: (0, i)),
        pl.BlockSpec((S, _B_BLK), lambda i: (0, i)),
        pl.BlockSpec((OTHER, _B_BLK), lambda i: (0, i)),
        _full((S, 256)),
        _full((S, 256)),
        _full((S, 256)),
        _full((OTHER, 256)),
        _full((1, 256)),
        _full((256, 128)),
        _full((1, 128)),
        _full((128, 128)),
        _full((1, 128)),
    ],
    out_specs=pl.BlockSpec((_B_BLK, 128), lambda i: (i, 0)),
    out_shape=jax.ShapeDtypeStruct((B, 128), jnp.float32),
)


def kernel(embedding, gene_with_go_idx, gene_with_go_value,
           gene_without_go_value, other_info, W1, b1, W2, b2, W3, b3):
    sums = _rowsum_call(embedding.T)
    idx_flat = gene_with_go_idx.T.reshape(BS).astype(jnp.int32)
    gathered = _sc_gather_call()(sums, idx_flat)
    g_t = gathered.reshape(S, B)
    w1a = W1[0:S]
    w1b = W1[S:2 * S]
    w1c = W1[2 * S:3 * S]
    w1d = W1[3 * S:]
    return _mlp_call(
        g_t, gene_with_go_value.T, gene_without_go_value.T, other_info.T,
        w1a, w1b, w1c, w1d, b1.reshape(1, 256),
        W2, b2.reshape(1, 128), W3, b3.reshape(1, 128))


# speedup vs baseline: 1.1356x; 1.1356x over previous
import functools

import jax
import jax.numpy as jnp
import numpy as np
from jax import lax
from jax.experimental import pallas as pl
from jax.experimental.pallas import tpu as pltpu
from jax.experimental.pallas import tpu_sc as plsc

B, S, D, V = 4096, 200, 64, 1000000
BS = B * S
OTHER = 64


_COLS_BLK = 32768
_N_BLKS = -(-V // _COLS_BLK)
_VPAD = _N_BLKS * _COLS_BLK


def _rowsum_body(emb_ref, out_ref):
    out_ref[...] = jnp.sum(emb_ref[...], axis=0)


_rowsum_call = pl.pallas_call(
    _rowsum_body,
    grid=(_N_BLKS,),
    in_specs=[pl.BlockSpec((D, _COLS_BLK), lambda i: (0, i))],
    out_specs=pl.BlockSpec((_COLS_BLK,), lambda i: (i,)),
    out_shape=jax.ShapeDtypeStruct((_VPAD,), jnp.float32),
)


_NC, _NS = 2, 16
_NW = _NC * _NS
_N_PER = BS // _NW


_N_CHUNKS = 4
_CHUNK = _N_PER // _N_CHUNKS


_SEG = None


def _sc_gather_body(sums_hbm, idx_hbm, out_hbm, shared, idx_v, vals_v, sems):
    cid = lax.axis_index("c")
    sid = lax.axis_index("s")
    wid = sid * _NC + cid
    base = wid * _N_PER
    seg = _VPAD // _NS
    pltpu.sync_copy(idx_hbm.at[pl.ds(base, _N_PER)], idx_v)
    pltpu.sync_copy(sums_hbm.at[pl.ds(sid * seg, seg)],
                    shared.at[pl.ds(sid * seg, seg)])
    plsc.subcore_barrier()
    copies = []
    for c in range(_N_CHUNKS):
        lo = c * _CHUNK
        copies.append(pltpu.async_copy(
            shared.at[idx_v.at[pl.ds(lo, _CHUNK)]],
            vals_v.at[pl.ds(lo, _CHUNK)], sems.at[c]))
    for c in range(_N_CHUNKS):
        lo = c * _CHUNK
        copies[c].wait()
        pltpu.sync_copy(vals_v.at[pl.ds(lo, _CHUNK)],
                        out_hbm.at[pl.ds(base + lo, _CHUNK)])


@functools.cache
def _sc_gather_call():
    return pl.kernel(
        _sc_gather_body,
        out_type=jax.ShapeDtypeStruct((BS,), jnp.float32),
        mesh=plsc.VectorSubcoreMesh(
            core_axis_name="c", subcore_axis_name="s", num_cores=_NC,
            num_subcores=_NS),
        scratch_types=[
            pltpu.VMEM_SHARED((_VPAD,), jnp.float32),
            pltpu.VMEM((_N_PER,), jnp.int32),
            pltpu.VMEM((_N_PER,), jnp.float32),
            pltpu.SemaphoreType.DMA((_N_CHUNKS,)),
        ],
    )


_B_BLK = 512


def _tdot(a_t, w):
    return lax.dot_general(a_t, w, (((0,), (0,)), ((), ())),
                           preferred_element_type=jnp.float32)


def _mlp_body(g_ref, gwv_ref, gwov_ref, oth_ref, w1a_ref, w1b_ref, w1c_ref,
              w1d_ref, b1_ref, w2_ref, b2_ref, w3_ref, b3_ref, out_ref):
    gwv_t = gwv_ref[...]
    emb_t = g_ref[...] * gwv_t * (1.0 / D)
    h = (_tdot(emb_t, w1a_ref[...]) + _tdot(gwv_t, w1b_ref[...])
         + _tdot(gwov_ref[...], w1c_ref[...]) + _tdot(oth_ref[...], w1d_ref[...])
         + b1_ref[...])
    h = jnp.maximum(h, 0.0)
    h = jnp.maximum(
        jnp.dot(h, w2_ref[...], preferred_element_type=jnp.float32) + b2_ref[...],
        0.0)
    out_ref[...] = (
        jnp.dot(h, w3_ref[...], preferred_element_type=jnp.float32) + b3_ref[...])


def _full(shape):
    return pl.BlockSpec(shape, lambda i: tuple(0 for _ in shape))


_mlp_call = pl.pallas_call(
    _mlp_body,
    grid=(B // _B_BLK,),
    in_specs=[
        pl.BlockSpec((S, _B_BLK), lambda i: (0, i)),
        pl.BlockSpec((S, _B_BLK), lambda i: (0, i)),
        pl.BlockSpec((S, _B_BLK), lambda i: (0, i)),
        pl.BlockSpec((OTHER, _B_BLK), lambda i: (0, i)),
        _full((S, 256)),
        _full((S, 256)),
        _full((S, 256)),
        _full((OTHER, 256)),
        _full((1, 256)),
        _full((256, 128)),
        _full((1, 128)),
        _full((128, 128)),
        _full((1, 128)),
    ],
    out_specs=pl.BlockSpec((_B_BLK, 128), lambda i: (i, 0)),
    out_shape=jax.ShapeDtypeStruct((B, 128), jnp.float32),
)


def kernel(embedding, gene_with_go_idx, gene_with_go_value,
           gene_without_go_value, other_info, W1, b1, W2, b2, W3, b3):
    sums = _rowsum_call(embedding.T)
    idx_flat = gene_with_go_idx.T.reshape(BS).astype(jnp.int32)
    gathered = _sc_gather_call()(sums, idx_flat)
    g_t = gathered.reshape(S, B)
    w1a = W1[0:S]
    w1b = W1[S:2 * S]
    w1c = W1[2 * S:3 * S]
    w1d = W1[3 * S:]
    return _mlp_call(
        g_t, gene_with_go_value.T, gene_without_go_value.T, other_info.T,
        w1a, w1b, w1c, w1d, b1.reshape(1, 256),
        W2, b2.reshape(1, 128), W3, b3.reshape(1, 128))


# speedup vs baseline: 1.1573x; 1.0191x over previous
import functools

import jax
import jax.numpy as jnp
import numpy as np
from jax import lax
from jax.experimental import pallas as pl
from jax.experimental.pallas import tpu as pltpu
from jax.experimental.pallas import tpu_sc as plsc

B, S, D, V = 4096, 200, 64, 1000000
BS = B * S
OTHER = 64


_COLS_BLK = 32768
_N_BLKS = -(-V // _COLS_BLK)
_VPAD = _N_BLKS * _COLS_BLK


def _rowsum_body(emb_ref, out_ref):
    out_ref[...] = jnp.sum(emb_ref[...], axis=0)


_rowsum_call = pl.pallas_call(
    _rowsum_body,
    grid=(_N_BLKS,),
    in_specs=[pl.BlockSpec((D, _COLS_BLK), lambda i: (0, i))],
    out_specs=pl.BlockSpec((_COLS_BLK,), lambda i: (i,)),
    out_shape=jax.ShapeDtypeStruct((_VPAD,), jnp.float32),
)


_NC, _NS = 2, 16
_NW = _NC * _NS
_N_PER = BS // _NW


_N_CHUNKS = 4
_CHUNK = _N_PER // _N_CHUNKS


_SEG = None


def _sc_gather_body(sums_hbm, idx_hbm, out_hbm, shared, idx_v, vals_v, sems):
    cid = lax.axis_index("c")
    sid = lax.axis_index("s")
    wid = sid * _NC + cid
    base = wid * _N_PER
    seg = _VPAD // _NS
    pltpu.sync_copy(idx_hbm.at[pl.ds(base, _N_PER)], idx_v)
    pltpu.sync_copy(sums_hbm.at[pl.ds(sid * seg, seg)],
                    shared.at[pl.ds(sid * seg, seg)])
    plsc.subcore_barrier()
    copies = []
    for c in range(_N_CHUNKS):
        lo = c * _CHUNK
        copies.append(pltpu.async_copy(
            shared.at[idx_v.at[pl.ds(lo, _CHUNK)]],
            vals_v.at[pl.ds(lo, _CHUNK)], sems.at[c]))
    for c in range(_N_CHUNKS):
        lo = c * _CHUNK
        copies[c].wait()
        pltpu.sync_copy(vals_v.at[pl.ds(lo, _CHUNK)],
                        out_hbm.at[pl.ds(base + lo, _CHUNK)])


@functools.cache
def _sc_gather_call():
    return pl.kernel(
        _sc_gather_body,
        out_type=jax.ShapeDtypeStruct((BS,), jnp.float32),
        mesh=plsc.VectorSubcoreMesh(
            core_axis_name="c", subcore_axis_name="s", num_cores=_NC,
            num_subcores=_NS),
        scratch_types=[
            pltpu.VMEM_SHARED((_VPAD,), jnp.float32),
            pltpu.VMEM((_N_PER,), jnp.int32),
            pltpu.VMEM((_N_PER,), jnp.float32),
            pltpu.SemaphoreType.DMA((_N_CHUNKS,)),
        ],
    )


_B_BLK = 1024


def _tdot(a_t, w):
    return lax.dot_general(a_t, w, (((0,), (0,)), ((), ())),
                           preferred_element_type=jnp.float32)


def _mlp_body(g_ref, gwv_ref, gwov_ref, oth_ref, w1a_ref, w1b_ref, w1c_ref,
              w1d_ref, b1_ref, w2_ref, b2_ref, w3_ref, b3_ref, out_ref):
    gwv_t = gwv_ref[...]
    emb_t = g_ref[...] * gwv_t * (1.0 / D)
    h = (_tdot(emb_t, w1a_ref[...]) + _tdot(gwv_t, w1b_ref[...])
         + _tdot(gwov_ref[...], w1c_ref[...]) + _tdot(oth_ref[...], w1d_ref[...])
         + b1_ref[...])
    h = jnp.maximum(h, 0.0)
    h = jnp.maximum(
        jnp.dot(h, w2_ref[...], preferred_element_type=jnp.float32) + b2_ref[...],
        0.0)
    out_ref[...] = (
        jnp.dot(h, w3_ref[...], preferred_element_type=jnp.float32) + b3_ref[...])


def _full(shape):
    return pl.BlockSpec(shape, lambda i: tuple(0 for _ in shape))


_mlp_call = pl.pallas_call(
    _mlp_body,
    grid=(B // _B_BLK,),
    in_specs=[
        pl.BlockSpec((S, _B_BLK), lambda i: (0, i)),
        pl.BlockSpec((S, _B_BLK), lambda i: (0, i)),
        pl.BlockSpec((S, _B_BLK), lambda i: (0, i)),
        pl.BlockSpec((OTHER, _B_BLK), lambda i: (0, i)),
        _full((S, 256)),
        _full((S, 256)),
        _full((S, 256)),
        _full((OTHER, 256)),
        _full((1, 256)),
        _full((256, 128)),
        _full((1, 128)),
        _full((128, 128)),
        _full((1, 128)),
    ],
    out_specs=pl.BlockSpec((_B_BLK, 128), lambda i: (i, 0)),
    out_shape=jax.ShapeDtypeStruct((B, 128), jnp.float32),
)


def kernel(embedding, gene_with_go_idx, gene_with_go_value,
           gene_without_go_value, other_info, W1, b1, W2, b2, W3, b3):
    sums = _rowsum_call(embedding.T)
    idx_flat = gene_with_go_idx.T.reshape(BS).astype(jnp.int32)
    gathered = _sc_gather_call()(sums, idx_flat)
    g_t = gathered.reshape(S, B)
    w1a = W1[0:S]
    w1b = W1[S:2 * S]
    w1c = W1[2 * S:3 * S]
    w1d = W1[3 * S:]
    return _mlp_call(
        g_t, gene_with_go_value.T, gene_without_go_value.T, other_info.T,
        w1a, w1b, w1c, w1d, b1.reshape(1, 256),
        W2, b2.reshape(1, 128), W3, b3.reshape(1, 128))


# speedup vs baseline: 1.1604x; 1.0027x over previous
import functools

import jax
import jax.numpy as jnp
import numpy as np
from jax import lax
from jax.experimental import pallas as pl
from jax.experimental.pallas import tpu as pltpu
from jax.experimental.pallas import tpu_sc as plsc

B, S, D, V = 4096, 200, 64, 1000000
BS = B * S
OTHER = 64


_COLS_BLK = 40960
_N_BLKS = -(-V // _COLS_BLK)
_VPAD = _N_BLKS * _COLS_BLK


def _rowsum_body(emb_ref, out_ref):
    out_ref[...] = jnp.sum(emb_ref[...], axis=0)


_rowsum_call = pl.pallas_call(
    _rowsum_body,
    grid=(_N_BLKS,),
    in_specs=[pl.BlockSpec((D, _COLS_BLK), lambda i: (0, i))],
    out_specs=pl.BlockSpec((_COLS_BLK,), lambda i: (i,)),
    out_shape=jax.ShapeDtypeStruct((_VPAD,), jnp.float32),
)


_NC, _NS = 2, 16
_NW = _NC * _NS
_N_PER = BS // _NW


_N_CHUNKS = 4
_CHUNK = _N_PER // _N_CHUNKS


_SEG = None


def _sc_gather_body(sums_hbm, idx_hbm, out_hbm, shared, idx_v, vals_v, sems):
    cid = lax.axis_index("c")
    sid = lax.axis_index("s")
    wid = sid * _NC + cid
    base = wid * _N_PER
    seg = _VPAD // _NS
    pltpu.sync_copy(idx_hbm.at[pl.ds(base, _N_PER)], idx_v)
    pltpu.sync_copy(sums_hbm.at[pl.ds(sid * seg, seg)],
                    shared.at[pl.ds(sid * seg, seg)])
    plsc.subcore_barrier()
    copies = []
    for c in range(_N_CHUNKS):
        lo = c * _CHUNK
        copies.append(pltpu.async_copy(
            shared.at[idx_v.at[pl.ds(lo, _CHUNK)]],
            vals_v.at[pl.ds(lo, _CHUNK)], sems.at[c]))
    for c in range(_N_CHUNKS):
        lo = c * _CHUNK
        copies[c].wait()
        pltpu.sync_copy(vals_v.at[pl.ds(lo, _CHUNK)],
                        out_hbm.at[pl.ds(base + lo, _CHUNK)])


@functools.cache
def _sc_gather_call():
    return pl.kernel(
        _sc_gather_body,
        out_type=jax.ShapeDtypeStruct((BS,), jnp.float32),
        mesh=plsc.VectorSubcoreMesh(
            core_axis_name="c", subcore_axis_name="s", num_cores=_NC,
            num_subcores=_NS),
        scratch_types=[
            pltpu.VMEM_SHARED((_VPAD,), jnp.float32),
            pltpu.VMEM((_N_PER,), jnp.int32),
            pltpu.VMEM((_N_PER,), jnp.float32),
            pltpu.SemaphoreType.DMA((_N_CHUNKS,)),
        ],
    )


_B_BLK = 2048


def _tdot(a_t, w):
    return lax.dot_general(a_t, w, (((0,), (0,)), ((), ())),
                           preferred_element_type=jnp.float32)


def _mlp_body(g_ref, gwv_ref, gwov_ref, oth_ref, w1a_ref, w1b_ref, w1c_ref,
              w1d_ref, b1_ref, w2_ref, b2_ref, w3_ref, b3_ref, out_ref):
    gwv_t = gwv_ref[...]
    emb_t = g_ref[...] * gwv_t * (1.0 / D)
    h = (_tdot(emb_t, w1a_ref[...]) + _tdot(gwv_t, w1b_ref[...])
         + _tdot(gwov_ref[...], w1c_ref[...]) + _tdot(oth_ref[...], w1d_ref[...])
         + b1_ref[...])
    h = jnp.maximum(h, 0.0)
    h = jnp.maximum(
        jnp.dot(h, w2_ref[...], preferred_element_type=jnp.float32) + b2_ref[...],
        0.0)
    out_ref[...] = (
        jnp.dot(h, w3_ref[...], preferred_element_type=jnp.float32) + b3_ref[...])


def _full(shape):
    return pl.BlockSpec(shape, lambda i: tuple(0 for _ in shape))


_mlp_call = pl.pallas_call(
    _mlp_body,
    grid=(B // _B_BLK,),
    in_specs=[
        pl.BlockSpec((S, _B_BLK), lambda i: (0, i)),
        pl.BlockSpec((S, _B_BLK), lambda i: (0, i)),
        pl.BlockSpec((S, _B_BLK), lambda i: (0, i)),
        pl.BlockSpec((OTHER, _B_BLK), lambda i: (0, i)),
        _full((S, 256)),
        _full((S, 256)),
        _full((S, 256)),
        _full((OTHER, 256)),
        _full((1, 256)),
        _full((256, 128)),
        _full((1, 128)),
        _full((128, 128)),
        _full((1, 128)),
    ],
    out_specs=pl.BlockSpec((_B_BLK, 128), lambda i: (i, 0)),
    out_shape=jax.ShapeDtypeStruct((B, 128), jnp.float32),
)


def kernel(embedding, gene_with_go_idx, gene_with_go_value,
           gene_without_go_value, other_info, W1, b1, W2, b2, W3, b3):
    sums = _rowsum_call(embedding.T)
    idx_flat = gene_with_go_idx.T.reshape(BS).astype(jnp.int32)
    gathered = _sc_gather_call()(sums, idx_flat)
    g_t = gathered.reshape(S, B)
    w1a = W1[0:S]
    w1b = W1[S:2 * S]
    w1c = W1[2 * S:3 * S]
    w1d = W1[3 * S:]
    return _mlp_call(
        g_t, gene_with_go_value.T, gene_without_go_value.T, other_info.T,
        w1a, w1b, w1c, w1d, b1.reshape(1, 256),
        W2, b2.reshape(1, 128), W3, b3.reshape(1, 128))


# speedup vs baseline: 1.2348x; 1.0641x over previous
import functools

import jax
import jax.numpy as jnp
import numpy as np
from jax import lax
from jax.experimental import pallas as pl
from jax.experimental.pallas import tpu as pltpu
from jax.experimental.pallas import tpu_sc as plsc

B, S, D, V = 4096, 200, 64, 1000000
BS = B * S
OTHER = 64


_COLS_BLK = 40960
_N_BLKS = -(-V // _COLS_BLK)
_VPAD = _N_BLKS * _COLS_BLK


def _rowsum_body(emb_ref, out_ref):
    out_ref[...] = jnp.sum(emb_ref[...], axis=0)


_rowsum_call = pl.pallas_call(
    _rowsum_body,
    grid=(_N_BLKS,),
    in_specs=[pl.BlockSpec((D, _COLS_BLK), lambda i: (0, i))],
    out_specs=pl.BlockSpec((_COLS_BLK,), lambda i: (i,)),
    out_shape=jax.ShapeDtypeStruct((_VPAD,), jnp.float32),
)


_NC, _NS = 2, 16
_NW = _NC * _NS
_N_PER = BS // _NW


_N_CHUNKS = 4
_CHUNK = _N_PER // _N_CHUNKS


_SEG = None


def _sc_gather_body(sums_hbm, idx_hbm, out_hbm, shared, idx_v, vals_v, sems):
    cid = lax.axis_index("c")
    sid = lax.axis_index("s")
    wid = sid * _NC + cid
    base = wid * _N_PER
    seg = _VPAD // _NS
    pltpu.sync_copy(idx_hbm.at[pl.ds(base, _N_PER)], idx_v)
    pltpu.sync_copy(sums_hbm.at[pl.ds(sid * seg, seg)],
                    shared.at[pl.ds(sid * seg, seg)])
    plsc.subcore_barrier()
    copies = []
    for c in range(_N_CHUNKS):
        lo = c * _CHUNK
        copies.append(pltpu.async_copy(
            shared.at[idx_v.at[pl.ds(lo, _CHUNK)]],
            vals_v.at[pl.ds(lo, _CHUNK)], sems.at[c]))
    for c in range(_N_CHUNKS):
        lo = c * _CHUNK
        copies[c].wait()
        pltpu.sync_copy(vals_v.at[pl.ds(lo, _CHUNK)],
                        out_hbm.at[pl.ds(base + lo, _CHUNK)])


@functools.cache
def _sc_gather_call():
    return pl.kernel(
        _sc_gather_body,
        out_type=jax.ShapeDtypeStruct((BS,), jnp.float32),
        mesh=plsc.VectorSubcoreMesh(
            core_axis_name="c", subcore_axis_name="s", num_cores=_NC,
            num_subcores=_NS),
        scratch_types=[
            pltpu.VMEM_SHARED((_VPAD,), jnp.float32),
            pltpu.VMEM((_N_PER,), jnp.int32),
            pltpu.VMEM((_N_PER,), jnp.float32),
            pltpu.SemaphoreType.DMA((_N_CHUNKS,)),
        ],
    )


_B_BLK = 2048


def _tdot(a_t, w):
    return lax.dot_general(a_t, w, (((0,), (0,)), ((), ())),
                           preferred_element_type=jnp.float32)


def _mlp_body(g_ref, gwv_ref, gwov_ref, oth_ref, w1a_ref, w1b_ref, w1c_ref,
              w1d_ref, b1_ref, w2_ref, b2_ref, w3_ref, b3_ref, out_ref):
    gwv_t = gwv_ref[...]
    emb_t = g_ref[...] * gwv_t * (1.0 / D)
    h = (_tdot(emb_t, w1a_ref[...]) + _tdot(gwv_t, w1b_ref[...])
         + _tdot(gwov_ref[...], w1c_ref[...]) + _tdot(oth_ref[...], w1d_ref[...])
         + b1_ref[...])
    h = jnp.maximum(h, 0.0)
    h = jnp.maximum(
        jnp.dot(h, w2_ref[...], preferred_element_type=jnp.float32) + b2_ref[...],
        0.0)
    out_ref[...] = (
        jnp.dot(h, w3_ref[...], preferred_element_type=jnp.float32) + b3_ref[...])


def _full(shape):
    return pl.BlockSpec(shape, lambda i: tuple(0 for _ in shape))


_mlp_call = pl.pallas_call(
    _mlp_body,
    grid=(B // _B_BLK,),
    in_specs=[
        pl.BlockSpec((S, _B_BLK), lambda i: (0, i)),
        pl.BlockSpec((S, _B_BLK), lambda i: (0, i)),
        pl.BlockSpec((S, _B_BLK), lambda i: (0, i)),
        pl.BlockSpec((OTHER, _B_BLK), lambda i: (0, i)),
        _full((S, 256)),
        _full((S, 256)),
        _full((S, 256)),
        _full((OTHER, 256)),
        _full((1, 256)),
        _full((256, 128)),
        _full((1, 128)),
        _full((128, 128)),
        _full((1, 128)),
    ],
    out_specs=pl.BlockSpec((_B_BLK, 128), lambda i: (i, 0)),
    out_shape=jax.ShapeDtypeStruct((B, 128), jnp.float32),
)


def kernel(embedding, gene_with_go_idx, gene_with_go_value,
           gene_without_go_value, other_info, W1, b1, W2, b2, W3, b3):
    sums = _rowsum_call(embedding.T)
    idx_t = gene_with_go_idx.T.astype(jnp.int32)
    idx_flat = (idx_t.reshape(S // 8, 8, B // 128, 128)
                .transpose(0, 2, 1, 3).reshape(BS))
    gathered = _sc_gather_call()(sums, idx_flat)
    g_t = (gathered.reshape(S // 8, B // 128, 8, 128)
           .transpose(0, 2, 1, 3).reshape(S, B))
    w1a = W1[0:S]
    w1b = W1[S:2 * S]
    w1c = W1[2 * S:3 * S]
    w1d = W1[3 * S:]
    return _mlp_call(
        g_t, gene_with_go_value.T, gene_without_go_value.T, other_info.T,
        w1a, w1b, w1c, w1d, b1.reshape(1, 256),
        W2, b2.reshape(1, 128), W3, b3.reshape(1, 128))


# speedup vs baseline: 1.2532x; 1.0149x over previous
import functools

import jax
import jax.numpy as jnp
import numpy as np
from jax import lax
from jax.experimental import pallas as pl
from jax.experimental.pallas import tpu as pltpu
from jax.experimental.pallas import tpu_sc as plsc

B, S, D, V = 4096, 200, 64, 1000000
BS = B * S
OTHER = 64


_COLS_BLK = 40960
_N_BLKS = -(-V // _COLS_BLK)
_VPAD = _N_BLKS * _COLS_BLK


def _rowsum_body(emb_ref, out_ref):
    out_ref[...] = jnp.sum(emb_ref[...], axis=0)


_rowsum_call = pl.pallas_call(
    _rowsum_body,
    grid=(_N_BLKS,),
    in_specs=[pl.BlockSpec((D, _COLS_BLK), lambda i: (0, i))],
    out_specs=pl.BlockSpec((_COLS_BLK,), lambda i: (i,)),
    out_shape=jax.ShapeDtypeStruct((_VPAD,), jnp.float32),
)


_NC, _NS = 2, 16
_NW = _NC * _NS
_N_PER = BS // _NW


_N_CHUNKS = 4
_CHUNK = _N_PER // _N_CHUNKS


_SEG = None


def _sc_gather_body(sums_hbm, idx_hbm, out_hbm, shared, idx_v, vals_v, sems):
    cid = lax.axis_index("c")
    sid = lax.axis_index("s")
    wid = sid * _NC + cid
    base = wid * _N_PER
    seg = _VPAD // _NS
    pltpu.sync_copy(idx_hbm.at[pl.ds(base, _N_PER)], idx_v)
    pltpu.sync_copy(sums_hbm.at[pl.ds(sid * seg, seg)],
                    shared.at[pl.ds(sid * seg, seg)])
    plsc.subcore_barrier()
    copies = []
    for c in range(_N_CHUNKS):
        lo = c * _CHUNK
        copies.append(pltpu.async_copy(
            shared.at[idx_v.at[pl.ds(lo, _CHUNK)]],
            vals_v.at[pl.ds(lo, _CHUNK)], sems.at[c]))
    for c in range(_N_CHUNKS):
        lo = c * _CHUNK
        copies[c].wait()
        pltpu.sync_copy(vals_v.at[pl.ds(lo, _CHUNK)],
                        out_hbm.at[pl.ds(base + lo, _CHUNK)])


@functools.cache
def _sc_gather_call():
    return pl.kernel(
        _sc_gather_body,
        out_type=jax.ShapeDtypeStruct((BS,), jnp.float32),
        mesh=plsc.VectorSubcoreMesh(
            core_axis_name="c", subcore_axis_name="s", num_cores=_NC,
            num_subcores=_NS),
        scratch_types=[
            pltpu.VMEM_SHARED((_VPAD,), jnp.float32),
            pltpu.VMEM((_N_PER,), jnp.int32),
            pltpu.VMEM((_N_PER,), jnp.float32),
            pltpu.SemaphoreType.DMA((_N_CHUNKS,)),
        ],
    )


_B_BLK = 2048


def _tdot(a_t, w):
    return lax.dot_general(a_t, w, (((0,), (0,)), ((), ())),
                           preferred_element_type=jnp.float32)


def _mlp_pre_body(gwv_ref, gwov_ref, oth_ref, w1b_ref, w1c_ref, w1d_ref,
                  b1_ref, out_ref):
    out_ref[...] = (_tdot(gwv_ref[...], w1b_ref[...])
                    + _tdot(gwov_ref[...], w1c_ref[...])
                    + _tdot(oth_ref[...], w1d_ref[...]) + b1_ref[...])


def _mlp_body(g_ref, gwv_ref, rest_ref, w1a_ref, w2_ref, b2_ref, w3_ref,
              b3_ref, out_ref):
    emb_t = g_ref[...] * gwv_ref[...] * (1.0 / D)
    h = jnp.maximum(_tdot(emb_t, w1a_ref[...]) + rest_ref[...], 0.0)
    h = jnp.maximum(
        jnp.dot(h, w2_ref[...], preferred_element_type=jnp.float32) + b2_ref[...],
        0.0)
    out_ref[...] = (
        jnp.dot(h, w3_ref[...], preferred_element_type=jnp.float32) + b3_ref[...])


def _full(shape):
    return pl.BlockSpec(shape, lambda i: tuple(0 for _ in shape))


_mlp_pre_call = pl.pallas_call(
    _mlp_pre_body,
    grid=(B // _B_BLK,),
    in_specs=[
        pl.BlockSpec((S, _B_BLK), lambda i: (0, i)),
        pl.BlockSpec((S, _B_BLK), lambda i: (0, i)),
        pl.BlockSpec((OTHER, _B_BLK), lambda i: (0, i)),
        _full((S, 256)),
        _full((S, 256)),
        _full((OTHER, 256)),
        _full((1, 256)),
    ],
    out_specs=pl.BlockSpec((_B_BLK, 256), lambda i: (i, 0)),
    out_shape=jax.ShapeDtypeStruct((B, 256), jnp.float32),
)

_mlp_call = pl.pallas_call(
    _mlp_body,
    grid=(B // _B_BLK,),
    in_specs=[
        pl.BlockSpec((S, _B_BLK), lambda i: (0, i)),
        pl.BlockSpec((S, _B_BLK), lambda i: (0, i)),
        pl.BlockSpec((_B_BLK, 256), lambda i: (i, 0)),
        _full((S, 256)),
        _full((256, 128)),
        _full((1, 128)),
        _full((128, 128)),
        _full((1, 128)),
    ],
    out_specs=pl.BlockSpec((_B_BLK, 128), lambda i: (i, 0)),
    out_shape=jax.ShapeDtypeStruct((B, 128), jnp.float32),
)


def kernel(embedding, gene_with_go_idx, gene_with_go_value,
           gene_without_go_value, other_info, W1, b1, W2, b2, W3, b3):
    sums = _rowsum_call(embedding.T)
    idx_t = gene_with_go_idx.T.astype(jnp.int32)
    idx_flat = (idx_t.reshape(S // 8, 8, B // 128, 128)
                .transpose(0, 2, 1, 3).reshape(BS))
    gathered = _sc_gather_call()(sums, idx_flat)
    g_t = (gathered.reshape(S // 8, B // 128, 8, 128)
           .transpose(0, 2, 1, 3).reshape(S, B))
    w1a = W1[0:S]
    w1b = W1[S:2 * S]
    w1c = W1[2 * S:3 * S]
    w1d = W1[3 * S:]
    rest = _mlp_pre_call(
        gene_with_go_value.T, gene_without_go_value.T, other_info.T,
        w1b, w1c, w1d, b1.reshape(1, 256))
    return _mlp_call(
        g_t, gene_with_go_value.T, rest, w1a,
        W2, b2.reshape(1, 128), W3, b3.reshape(1, 128))


# speedup vs baseline: 1.2535x; 1.0002x over previous
import functools

import jax
import jax.numpy as jnp
from jax import lax
from jax.experimental import pallas as pl
from jax.experimental.pallas import tpu as pltpu
from jax.experimental.pallas import tpu_sc as plsc

B, S, D, V = 4096, 200, 64, 1000000
BS = B * S
OTHER = 64


_COLS_BLK = 40960
_N_BLKS = -(-V // _COLS_BLK)
_VPAD = _N_BLKS * _COLS_BLK


def _rowsum_body(emb_ref, out_ref):
    out_ref[...] = jnp.sum(emb_ref[...], axis=0)


_rowsum_call = pl.pallas_call(
    _rowsum_body,
    grid=(_N_BLKS,),
    in_specs=[pl.BlockSpec((D, _COLS_BLK), lambda i: (0, i))],
    out_specs=pl.BlockSpec((_COLS_BLK,), lambda i: (i,)),
    out_shape=jax.ShapeDtypeStruct((_VPAD,), jnp.float32),
)


_NC, _NS = 2, 16
_NW = _NC * _NS
_N_PER = BS // _NW


_N_CHUNKS = 4
_CHUNK = _N_PER // _N_CHUNKS


def _sc_gather_body(sums_hbm, idx_hbm, out_hbm, shared, idx_v, vals_v, sems):
    cid = lax.axis_index("c")
    sid = lax.axis_index("s")
    wid = sid * _NC + cid
    base = wid * _N_PER
    seg = _VPAD // _NS
    pltpu.sync_copy(idx_hbm.at[pl.ds(base, _N_PER)], idx_v)
    pltpu.sync_copy(sums_hbm.at[pl.ds(sid * seg, seg)],
                    shared.at[pl.ds(sid * seg, seg)])
    plsc.subcore_barrier()
    copies = []
    for c in range(_N_CHUNKS):
        lo = c * _CHUNK
        copies.append(pltpu.async_copy(
            shared.at[idx_v.at[pl.ds(lo, _CHUNK)]],
            vals_v.at[pl.ds(lo, _CHUNK)], sems.at[c]))
    for c in range(_N_CHUNKS):
        lo = c * _CHUNK
        copies[c].wait()
        pltpu.sync_copy(vals_v.at[pl.ds(lo, _CHUNK)],
                        out_hbm.at[pl.ds(base + lo, _CHUNK)])


@functools.cache
def _sc_gather_call():
    return pl.kernel(
        _sc_gather_body,
        out_type=jax.ShapeDtypeStruct((BS,), jnp.float32),
        mesh=plsc.VectorSubcoreMesh(
            core_axis_name="c", subcore_axis_name="s", num_cores=_NC,
            num_subcores=_NS),
        scratch_types=[
            pltpu.VMEM_SHARED((_VPAD,), jnp.float32),
            pltpu.VMEM((_N_PER,), jnp.int32),
            pltpu.VMEM((_N_PER,), jnp.float32),
            pltpu.SemaphoreType.DMA((_N_CHUNKS,)),
        ],
    )


_B_BLK = 2048


def _tdot(a_t, w):
    return lax.dot_general(a_t, w, (((0,), (0,)), ((), ())),
                           preferred_element_type=jnp.float32)


def _mlp_pre_body(gwv_ref, gwov_ref, oth_ref, w1b_ref, w1c_ref, w1d_ref,
                  b1_ref, out_ref):
    out_ref[...] = (_tdot(gwv_ref[...], w1b_ref[...])
                    + _tdot(gwov_ref[...], w1c_ref[...])
                    + _tdot(oth_ref[...], w1d_ref[...]) + b1_ref[...])


def _mlp_body(g_ref, gwv_ref, rest_ref, w1a_ref, w2_ref, b2_ref, w3_ref,
              b3_ref, out_ref):
    emb_t = g_ref[...] * gwv_ref[...] * (1.0 / D)
    h = jnp.maximum(_tdot(emb_t, w1a_ref[...]) + rest_ref[...], 0.0)
    h = jnp.maximum(
        jnp.dot(h, w2_ref[...], preferred_element_type=jnp.float32) + b2_ref[...],
        0.0)
    out_ref[...] = (
        jnp.dot(h, w3_ref[...], preferred_element_type=jnp.float32) + b3_ref[...])


def _full(shape):
    return pl.BlockSpec(shape, lambda i: tuple(0 for _ in shape))


_mlp_pre_call = pl.pallas_call(
    _mlp_pre_body,
    grid=(B // _B_BLK,),
    in_specs=[
        pl.BlockSpec((S, _B_BLK), lambda i: (0, i)),
        pl.BlockSpec((S, _B_BLK), lambda i: (0, i)),
        pl.BlockSpec((OTHER, _B_BLK), lambda i: (0, i)),
        _full((S, 256)),
        _full((S, 256)),
        _full((OTHER, 256)),
        _full((1, 256)),
    ],
    out_specs=pl.BlockSpec((_B_BLK, 256), lambda i: (i, 0)),
    out_shape=jax.ShapeDtypeStruct((B, 256), jnp.float32),
)

_mlp_call = pl.pallas_call(
    _mlp_body,
    grid=(B // _B_BLK,),
    in_specs=[
        pl.BlockSpec((S, _B_BLK), lambda i: (0, i)),
        pl.BlockSpec((S, _B_BLK), lambda i: (0, i)),
        pl.BlockSpec((_B_BLK, 256), lambda i: (i, 0)),
        _full((S, 256)),
        _full((256, 128)),
        _full((1, 128)),
        _full((128, 128)),
        _full((1, 128)),
    ],
    out_specs=pl.BlockSpec((_B_BLK, 128), lambda i: (i, 0)),
    out_shape=jax.ShapeDtypeStruct((B, 128), jnp.float32),
)


def kernel(embedding, gene_with_go_idx, gene_with_go_value,
           gene_without_go_value, other_info, W1, b1, W2, b2, W3, b3):
    sums = _rowsum_call(embedding.T)
    idx_t = gene_with_go_idx.T.astype(jnp.int32)
    idx_flat = (idx_t.reshape(S // 8, 8, B // 128, 128)
                .transpose(0, 2, 1, 3).reshape(BS))
    gathered = _sc_gather_call()(sums, idx_flat)
    g_t = (gathered.reshape(S // 8, B // 128, 8, 128)
           .transpose(0, 2, 1, 3).reshape(S, B))
    w1a = W1[0:S]
    w1b = W1[S:2 * S]
    w1c = W1[2 * S:3 * S]
    w1d = W1[3 * S:]
    rest = _mlp_pre_call(
        gene_with_go_value.T, gene_without_go_value.T, other_info.T,
        w1b, w1c, w1d, b1.reshape(1, 256))
    return _mlp_call(
        g_t, gene_with_go_value.T, rest, w1a,
        W2, b2.reshape(1, 128), W3, b3.reshape(1, 128))
